# Initial kernel scaffold; baseline (speedup 1.0000x reference)
import jax
import jax.numpy as jnp
from jax.experimental import pallas as pl


def kernel(X, embeddings, bias, w0):
    raise NotImplementedError("write your pallas kernel here")


# speedup vs baseline: 1.1736x; 1.1736x over previous
import functools

import jax
import jax.numpy as jnp
from jax import lax
from jax.experimental import pallas as pl
from jax.experimental.pallas import tpu as pltpu
from jax.experimental.pallas import tpu_sc as plsc

N_VOCAB = 1000000
K = 16
BATCH = 16384
FIELDS = 26

NC = 2
NS = 16
NW = NC * NS
SAMPLES_PER_W = BATCH // NW
CHUNK = 128
N_CHUNKS = SAMPLES_PER_W // CHUNK
IDX_ROWS = CHUNK * FIELDS // 128


def _fm_body(x_hbm, emb_hbm, bias_hbm, w0_hbm, out_hbm,
             idx_v, emb_v, bias_v, out_v, w0_v, esem, bsem):
    wid = lax.axis_index("s") * NC + lax.axis_index("c")

    pltpu.sync_copy(w0_hbm, w0_v)
    w0vec = w0_v[...]

    iota = lax.iota(jnp.int32, 16)
    iota26 = iota * FIELDS

    for c in range(N_CHUNKS):
        idx_off = (wid * N_CHUNKS + c) * (CHUNK * FIELDS)
        pltpu.sync_copy(x_hbm.at[pl.ds(idx_off, CHUNK * FIELDS)], idx_v)

        descs = []
        for j in range(IDX_ROWS):
            descs.append(pltpu.async_copy(
                emb_hbm.at[idx_v.at[pl.ds(j * 128, 128)]],
                emb_v.at[pl.ds(j * 128, 128)], esem))
            descs.append(pltpu.async_copy(
                bias_hbm.at[idx_v.at[pl.ds(j * 128, 128)]],
                bias_v.at[pl.ds(j * 128, 128)], bsem))
        for d in descs:
            d.wait()

        def group_body(g, _):
            rbase = iota26 + g * (16 * FIELDS)

            def k_body(k, acc):
                cols = jnp.full((16,), k, jnp.int32)
                s = jnp.zeros((16,), jnp.float32)
                q = jnp.zeros((16,), jnp.float32)
                for f in range(FIELDS):
                    v = plsc.load_gather(emb_v, [rbase + f, cols])
                    s = s + v
                    q = q + v * v
                return acc + (s * s - q)

            pair = lax.fori_loop(0, K, k_body, jnp.zeros((16,), jnp.float32))
            bacc = jnp.zeros((16,), jnp.float32)
            for f in range(FIELDS):
                bacc = bacc + plsc.load_gather(bias_v, [rbase + f])
            t = w0vec + bacc + 0.5 * pair
            out_v[pl.ds(g * 16, 16)] = 5.5 / (1.0 + jnp.exp(-t))
            return 0

        lax.fori_loop(0, CHUNK // 16, group_body, 0)

        out_off = wid * SAMPLES_PER_W + c * CHUNK
        pltpu.sync_copy(out_v, out_hbm.at[pl.ds(out_off, CHUNK)])


@jax.jit
def _fm_call(x2d, emb, bias, w0b):
    mesh = plsc.VectorSubcoreMesh(core_axis_name="c", subcore_axis_name="s")
    return pl.kernel(
        _fm_body,
        out_type=jax.ShapeDtypeStruct((BATCH,), jnp.float32),
        mesh=mesh,
        scratch_types=[
            pltpu.VMEM((CHUNK * FIELDS,), jnp.int32),
            pltpu.VMEM((CHUNK * FIELDS, K), jnp.float32),
            pltpu.VMEM((CHUNK * FIELDS,), jnp.float32),
            pltpu.VMEM((CHUNK,), jnp.float32),
            pltpu.VMEM((16,), jnp.float32),
            pltpu.SemaphoreType.DMA,
            pltpu.SemaphoreType.DMA,
        ],
        compiler_params=pltpu.CompilerParams(
            needs_layout_passes=False, use_tc_tiling_on_sc=False),
    )(x2d, emb, bias, w0b)


def kernel(X, embeddings, bias, w0):
    xflat = X.astype(jnp.int32).reshape(BATCH * FIELDS)
    w0b = jnp.broadcast_to(w0.astype(jnp.float32), (16,))
    return _fm_call(xflat, embeddings, bias.reshape(N_VOCAB), w0b)
